# baseline (speedup 1.0000x reference)
import functools

import jax
import jax.numpy as jnp
from jax import lax
from jax.experimental import pallas as pl
from jax.experimental.pallas import tpu as pltpu
from jax.experimental.pallas import tpu_sc as plsc

N = 10000
E = 320000
D = 128
H = 128

NC = 2
NS = 16
L = 16

R = N // NS
EC = E // NC
C = 2000
NCHUNK = EC // C
K = 64
RING = 4096
RMASK = RING - 1

NEG_INF = float("-inf")


def _tc1_body(x_ref, wf_ref, bf_ref, wih_ref, brnn_ref, wg_ref,
              g_ref, fi_ref):
    feat = jnp.tanh(
        jnp.dot(x_ref[...], wf_ref[...], preferred_element_type=jnp.float32)
        + bf_ref[...])
    fi = jnp.dot(feat, wih_ref[...], preferred_element_type=jnp.float32)
    h0 = jnp.tanh(fi + brnn_ref[...])
    g_ref[...] = jnp.dot(h0, wg_ref[...],
                         preferred_element_type=jnp.float32)
    fi_ref[...] = fi


def _tc1(x, W_f, b_f, W_ih, b_rnn, W_g):
    B = 1000
    grid = (N // B,)
    row_spec = pl.BlockSpec((B, H), lambda i: (i, 0))
    w_spec = pl.BlockSpec((H, H), lambda i: (0, 0))
    v_spec = pl.BlockSpec((1, H), lambda i: (0, 0))
    return pl.pallas_call(
        _tc1_body,
        grid=grid,
        in_specs=[pl.BlockSpec((B, D), lambda i: (i, 0)), w_spec, v_spec,
                  w_spec, v_spec, w_spec],
        out_specs=[row_spec, row_spec],
        out_shape=[jax.ShapeDtypeStruct((N, H), jnp.float32),
                   jax.ShapeDtypeStruct((N, H), jnp.float32)],
    )(x, W_f, b_f.reshape(1, H), W_ih, b_rnn.reshape(1, H), W_g)


def _sc_body(g_hbm, src_hbm, dst_hbm, out_hbm,
             acc, srcb0, dstb0, srcb1, dstb1, cidx, coff,
             rows0, rows1, stage_s, stage_o, se0, se1, sg0, sg1):
    cid = lax.axis_index("c")
    sid = lax.axis_index("s")
    lo = sid * R
    lane = lax.broadcasted_iota(jnp.int32, (L,), 0)
    ebase = cid * EC

    neg = jnp.full((L,), NEG_INF, dtype=jnp.float32)

    def init_body(i, _):
        acc[pl.ds(i * L, L)] = neg
        return 0

    lax.fori_loop(0, (R + 1) * H // L, init_body, 0)

    chunk_bufs = ((srcb0, dstb0, se0), (srcb1, dstb1, se1))

    def chunk_descs(ch, par):
        sb, db, se = chunk_bufs[par]
        b = pl.multiple_of(ebase + ch * C, 8)
        c1 = pltpu.make_async_copy(src_hbm.at[pl.ds(b, C)], sb, se)
        c2 = pltpu.make_async_copy(dst_hbm.at[pl.ds(b, C)], db, se)
        return c1, c2

    def fire_chunk(ch, par):
        c1, c2 = chunk_descs(ch, par)
        c1.start()
        c2.start()

    def wait_chunk(ch, par):
        c1, c2 = chunk_descs(ch, par)
        c1.wait()
        c2.wait()

    gather_bufs = ((rows0, sg0), (rows1, sg1))

    def gather_desc(dlog, p):
        rws, sg = gather_bufs[p]
        dphys = pl.multiple_of(dlog & RMASK, K)
        return pltpu.make_async_copy(
            g_hbm.at[cidx.at[pl.ds(dphys, K)]], rws, sg)

    def fire_gather(dlog):
        pb = (dlog // K) & 1

        @pl.when(pb == 0)
        def _():
            gather_desc(dlog, 0).start()

        @pl.when(pb == 1)
        def _():
            gather_desc(dlog, 1).start()

    SG = 4

    def _wait_acc(dlog, p):
        gather_desc(dlog, p).wait()
        rws = gather_bufs[p][0]
        dphys = pl.multiple_of(dlog & RMASK, K)

        def grp_body(jj, _):
            offv = coff[pl.ds(dphys + jj * L, L)]
            for t0 in range(0, L, SG):
                offs = [pl.multiple_of(offv[t0 + u], 8) for u in range(SG)]
                rs = [jj * L + t0 + u for u in range(SG)]
                dup = jnp.bool_(False)
                for a in range(SG):
                    for b in range(a + 1, SG):
                        dup = dup | (offs[a] == offs[b])

                @pl.when(jnp.logical_not(dup))
                def _fast():
                    vals = []
                    for u in range(SG):
                        for c in range(H // L):
                            sl = pl.ds(offs[u] + c * L, L)
                            vals.append(jnp.maximum(
                                acc[sl], rws[rs[u], pl.ds(c * L, L)]))
                    for u in range(SG):
                        for c in range(H // L):
                            sl = pl.ds(offs[u] + c * L, L)
                            acc[sl] = vals[u * (H // L) + c]

                @pl.when(dup)
                def _slow():
                    for u in range(SG):
                        for c in range(H // L):
                            sl = pl.ds(offs[u] + c * L, L)
                            acc[sl] = jnp.maximum(
                                acc[sl], rws[rs[u], pl.ds(c * L, L)])
            return 0

        lax.fori_loop(0, K // L, grp_body, 0)

    def acc_block(dlog):
        pb = (dlog // K) & 1

        @pl.when(pb == 0)
        def _():
            _wait_acc(dlog, 0)

        @pl.when(pb == 1)
        def _():
            _wait_acc(dlog, 1)

    def drains(cnt, dq, gq):
        def cond(carry):
            dq, gq = carry
            return cnt - dq >= K

        def body(carry):
            dq, gq = carry

            @pl.when(gq == dq)
            def _():
                fire_gather(dq)

            gq = jnp.maximum(gq, dq + K)
            ahead = cnt - gq >= K

            @pl.when(ahead)
            def _():
                fire_gather(gq)

            gq = jnp.where(ahead, gq + K, gq)
            acc_block(dq)
            return dq + K, gq

        return lax.while_loop(cond, body, (dq, gq))

    def filter_from(par, cnt_vec):
        sb, db, _ = chunk_bufs[par]

        def filt_body(i, cnt_vec):
            dv = db[pl.ds(i * L, L)]
            sv = sb[pl.ds(i * L, L)]
            m = (dv >= lo) & (dv < lo + R)
            mi = jnp.where(m, jnp.int32(1), jnp.int32(0))
            pref = plsc.cumsum(mi)
            pos = (cnt_vec + pref - 1) & RMASK
            plsc.store_scatter(cidx, [pos], sv, mask=m)
            plsc.store_scatter(coff, [pos], (dv - lo) * H, mask=m)
            return cnt_vec + plsc.all_reduce_population_count(m)

        return lax.fori_loop(0, C // L, filt_body, cnt_vec)

    fire_chunk(jnp.int32(0), 0)

    def pair_body(i, carry):
        cnt_vec, dq, gq = carry
        ch0 = i * 2
        wait_chunk(ch0, 0)
        fire_chunk(ch0 + 1, 1)
        cnt_vec = filter_from(0, cnt_vec)
        dq, gq = drains(cnt_vec[0], dq, gq)
        wait_chunk(ch0 + 1, 1)

        @pl.when(ch0 + 2 < NCHUNK)
        def _():
            fire_chunk(ch0 + 2, 0)

        cnt_vec = filter_from(1, cnt_vec)
        dq, gq = drains(cnt_vec[0], dq, gq)
        return cnt_vec, dq, gq

    cnt_vec, dq, gq = lax.fori_loop(
        0, NCHUNK // 2, pair_body,
        (jnp.zeros((L,), jnp.int32), jnp.int32(0), jnp.int32(0)))
    cnt = cnt_vec[0]

    pad_idx = lane + sid * L
    pad_off = jnp.full((L,), R * H, dtype=jnp.int32)
    for t in range(K // L):
        pos = (cnt + t * L + lane) & RMASK
        plsc.store_scatter(cidx, [pos], pad_idx)
        plsc.store_scatter(coff, [pos], pad_off)
    drains(dq + K, dq, gq)

    pltpu.sync_copy(acc.at[pl.ds(0, R * H)],
                    out_hbm.at[cid, pl.ds(pl.multiple_of(lo * H, 8), R * H)])


def _sc_segmax(g, src, dst):
    mesh = plsc.VectorSubcoreMesh(core_axis_name="c", subcore_axis_name="s")
    kern = pl.kernel(
        _sc_body,
        out_type=jax.ShapeDtypeStruct((NC, N * H), jnp.float32),
        mesh=mesh,
        compiler_params=pltpu.CompilerParams(needs_layout_passes=False),
        scratch_types=[
            pltpu.VMEM(((R + 1) * H,), jnp.float32),
            pltpu.VMEM((C,), jnp.int32),
            pltpu.VMEM((C,), jnp.int32),
            pltpu.VMEM((C,), jnp.int32),
            pltpu.VMEM((C,), jnp.int32),
            pltpu.VMEM((RING,), jnp.int32),
            pltpu.VMEM((RING,), jnp.int32),
            pltpu.VMEM((K, H), jnp.float32),
            pltpu.VMEM((K, H), jnp.float32),
            pltpu.VMEM((L,), jnp.int32),
            pltpu.VMEM((L,), jnp.int32),
            pltpu.SemaphoreType.DMA,
            pltpu.SemaphoreType.DMA,
            pltpu.SemaphoreType.DMA,
            pltpu.SemaphoreType.DMA,
        ],
    )
    return kern(g, src, dst)


def _tc2_body(fi_ref, a0_ref, a1_ref, whh_ref, brnn_ref, h_ref):
    m = jnp.maximum(a0_ref[...], a1_ref[...])
    m = jnp.where(m == NEG_INF, 0.0, m)
    h_ref[...] = jnp.tanh(
        fi_ref[...]
        + jnp.dot(m, whh_ref[...], preferred_element_type=jnp.float32)
        + brnn_ref[...])


def _tc2(fi, a0, a1, W_hh, b_rnn):
    B = 1000
    grid = (N // B,)
    row_spec = pl.BlockSpec((B, H), lambda i: (i, 0))
    w_spec = pl.BlockSpec((H, H), lambda i: (0, 0))
    v_spec = pl.BlockSpec((1, H), lambda i: (0, 0))
    return pl.pallas_call(
        _tc2_body,
        grid=grid,
        in_specs=[row_spec, row_spec, row_spec, w_spec, v_spec],
        out_specs=row_spec,
        out_shape=jax.ShapeDtypeStruct((N, H), jnp.float32),
    )(fi, a0, a1, W_hh, b_rnn.reshape(1, H))


def kernel(x, edge_index, W_f, b_f, W_g, W_ih, W_hh, b_rnn):
    g, fi = _tc1(x, W_f, b_f, W_ih, b_rnn, W_g)
    src = edge_index[0]
    dst = edge_index[1]
    agg2 = _sc_segmax(g, src, dst)
    a0 = agg2[0].reshape(N, H)
    a1 = agg2[1].reshape(N, H)
    return _tc2(fi, a0, a1, W_hh, b_rnn)


# speedup vs baseline: 1.1110x; 1.1110x over previous
import functools

import jax
import jax.numpy as jnp
from jax import lax
from jax.experimental import pallas as pl
from jax.experimental.pallas import tpu as pltpu
from jax.experimental.pallas import tpu_sc as plsc

N = 10000
E = 320000
D = 128
H = 128

NC = 2
NS = 16
L = 16

R = N // NS
EC = E // NC
C = 2000
NCHUNK = EC // C
K = 128
RING = 4096
RMASK = RING - 1

NEG_INF = float("-inf")


def _tc1_body(x_ref, wf_ref, bf_ref, wih_ref, brnn_ref, wg_ref,
              g_ref, fi_ref):
    feat = jnp.tanh(
        jnp.dot(x_ref[...], wf_ref[...], preferred_element_type=jnp.float32)
        + bf_ref[...])
    fi = jnp.dot(feat, wih_ref[...], preferred_element_type=jnp.float32)
    h0 = jnp.tanh(fi + brnn_ref[...])
    g_ref[...] = jnp.dot(h0, wg_ref[...],
                         preferred_element_type=jnp.float32)
    fi_ref[...] = fi


def _tc1(x, W_f, b_f, W_ih, b_rnn, W_g):
    B = 1000
    grid = (N // B,)
    row_spec = pl.BlockSpec((B, H), lambda i: (i, 0))
    w_spec = pl.BlockSpec((H, H), lambda i: (0, 0))
    v_spec = pl.BlockSpec((1, H), lambda i: (0, 0))
    return pl.pallas_call(
        _tc1_body,
        grid=grid,
        in_specs=[pl.BlockSpec((B, D), lambda i: (i, 0)), w_spec, v_spec,
                  w_spec, v_spec, w_spec],
        out_specs=[row_spec, row_spec],
        out_shape=[jax.ShapeDtypeStruct((N, H), jnp.float32),
                   jax.ShapeDtypeStruct((N, H), jnp.float32)],
    )(x, W_f, b_f.reshape(1, H), W_ih, b_rnn.reshape(1, H), W_g)


def _sc_body(g_hbm, src_hbm, dst_hbm, out_hbm,
             acc, srcb0, dstb0, srcb1, dstb1, cidx, coff,
             rows0, rows1, stage_s, stage_o, se0, se1, sg0, sg1):
    cid = lax.axis_index("c")
    sid = lax.axis_index("s")
    lo = sid * R
    lane = lax.broadcasted_iota(jnp.int32, (L,), 0)
    ebase = cid * EC

    neg = jnp.full((L,), NEG_INF, dtype=jnp.float32)

    def init_body(i, _):
        for u in range(8):
            acc[pl.ds((i * 8 + u) * L, L)] = neg
        return 0

    lax.fori_loop(0, (R + 1) * H // (L * 8), init_body, 0)
    for i in range(((R + 1) * H // (L * 8)) * 8, (R + 1) * H // L):
        acc[pl.ds(i * L, L)] = neg

    chunk_bufs = ((srcb0, dstb0, se0), (srcb1, dstb1, se1))

    def chunk_descs(ch, par):
        sb, db, se = chunk_bufs[par]
        b = pl.multiple_of(ebase + ch * C, 8)
        c1 = pltpu.make_async_copy(src_hbm.at[pl.ds(b, C)], sb, se)
        c2 = pltpu.make_async_copy(dst_hbm.at[pl.ds(b, C)], db, se)
        return c1, c2

    def fire_chunk(ch, par):
        c1, c2 = chunk_descs(ch, par)
        c1.start()
        c2.start()

    def wait_chunk(ch, par):
        c1, c2 = chunk_descs(ch, par)
        c1.wait()
        c2.wait()

    gather_bufs = ((rows0, sg0), (rows1, sg1))

    def gather_desc(dlog, p):
        rws, sg = gather_bufs[p]
        dphys = pl.multiple_of(dlog & RMASK, K)
        return pltpu.make_async_copy(
            g_hbm.at[cidx.at[pl.ds(dphys, K)]], rws, sg)

    def fire_gather(dlog):
        pb = (dlog // K) & 1

        @pl.when(pb == 0)
        def _():
            gather_desc(dlog, 0).start()

        @pl.when(pb == 1)
        def _():
            gather_desc(dlog, 1).start()

    SG = 4

    def _wait_acc(dlog, p):
        gather_desc(dlog, p).wait()
        rws = gather_bufs[p][0]
        dphys = pl.multiple_of(dlog & RMASK, K)

        def grp_body(jj, _):
            offv = coff[pl.ds(dphys + jj * L, L)]
            offs = [pl.multiple_of(offv[t], 8) for t in range(L)]
            dup = jnp.bool_(False)
            for t0 in range(0, L, SG):
                for a in range(SG):
                    for b in range(a + 1, SG):
                        dup = dup | (offs[t0 + a] == offs[t0 + b])

            @pl.when(jnp.logical_not(dup))
            def _fast():
                for t0 in range(0, L, SG):
                    vals = []
                    for u in range(SG):
                        r = jj * L + t0 + u
                        for c in range(H // L):
                            sl = pl.ds(offs[t0 + u] + c * L, L)
                            vals.append(jnp.maximum(
                                acc[sl], rws[r, pl.ds(c * L, L)]))
                    for u in range(SG):
                        for c in range(H // L):
                            sl = pl.ds(offs[t0 + u] + c * L, L)
                            acc[sl] = vals[u * (H // L) + c]

            @pl.when(dup)
            def _slow():
                for t in range(L):
                    r = jj * L + t
                    for c in range(H // L):
                        sl = pl.ds(offs[t] + c * L, L)
                        acc[sl] = jnp.maximum(
                            acc[sl], rws[r, pl.ds(c * L, L)])
            return 0

        lax.fori_loop(0, K // L, grp_body, 0)

    def acc_block(dlog):
        pb = (dlog // K) & 1

        @pl.when(pb == 0)
        def _():
            _wait_acc(dlog, 0)

        @pl.when(pb == 1)
        def _():
            _wait_acc(dlog, 1)

    def drains(cnt, dq, gq):
        def cond(carry):
            dq, gq = carry
            return cnt - dq >= K

        def body(carry):
            dq, gq = carry

            @pl.when(gq == dq)
            def _():
                fire_gather(dq)

            gq = jnp.maximum(gq, dq + K)
            ahead = cnt - gq >= K

            @pl.when(ahead)
            def _():
                fire_gather(gq)

            gq = jnp.where(ahead, gq + K, gq)
            acc_block(dq)
            return dq + K, gq

        return lax.while_loop(cond, body, (dq, gq))

    def filter_from(par, cnt_vec):
        sb, db, _ = chunk_bufs[par]

        FU = 5

        def filt_body(i, cnt_vec):
            for u in range(FU):
                dv = db[pl.ds((i * FU + u) * L, L)]
                sv = sb[pl.ds((i * FU + u) * L, L)]
                m = (dv >= lo) & (dv < lo + R)
                mi = jnp.where(m, jnp.int32(1), jnp.int32(0))
                pref = plsc.cumsum(mi)
                pos = (cnt_vec + pref - 1) & RMASK
                plsc.store_scatter(cidx, [pos], sv, mask=m)
                plsc.store_scatter(coff, [pos], (dv - lo) * H, mask=m)
                cnt_vec = cnt_vec + plsc.all_reduce_population_count(m)
            return cnt_vec

        return lax.fori_loop(0, C // L // FU, filt_body, cnt_vec)

    fire_chunk(jnp.int32(0), 0)

    def pair_body(i, carry):
        cnt_vec, dq, gq = carry
        ch0 = i * 2
        wait_chunk(ch0, 0)
        fire_chunk(ch0 + 1, 1)
        cnt_vec = filter_from(0, cnt_vec)
        dq, gq = drains(cnt_vec[0], dq, gq)
        wait_chunk(ch0 + 1, 1)

        @pl.when(ch0 + 2 < NCHUNK)
        def _():
            fire_chunk(ch0 + 2, 0)

        cnt_vec = filter_from(1, cnt_vec)
        dq, gq = drains(cnt_vec[0], dq, gq)
        return cnt_vec, dq, gq

    cnt_vec, dq, gq = lax.fori_loop(
        0, NCHUNK // 2, pair_body,
        (jnp.zeros((L,), jnp.int32), jnp.int32(0), jnp.int32(0)))
    cnt = cnt_vec[0]

    pad_idx = lane + sid * L
    pad_off = jnp.full((L,), R * H, dtype=jnp.int32)
    for t in range(K // L):
        pos = (cnt + t * L + lane) & RMASK
        plsc.store_scatter(cidx, [pos], pad_idx)
        plsc.store_scatter(coff, [pos], pad_off)
    drains(dq + K, dq, gq)

    pltpu.sync_copy(acc.at[pl.ds(0, R * H)],
                    out_hbm.at[cid, pl.ds(pl.multiple_of(lo * H, 8), R * H)])


def _sc_segmax(g, src, dst):
    mesh = plsc.VectorSubcoreMesh(core_axis_name="c", subcore_axis_name="s")
    kern = pl.kernel(
        _sc_body,
        out_type=jax.ShapeDtypeStruct((NC, N * H), jnp.float32),
        mesh=mesh,
        compiler_params=pltpu.CompilerParams(needs_layout_passes=False),
        scratch_types=[
            pltpu.VMEM(((R + 1) * H,), jnp.float32),
            pltpu.VMEM((C,), jnp.int32),
            pltpu.VMEM((C,), jnp.int32),
            pltpu.VMEM((C,), jnp.int32),
            pltpu.VMEM((C,), jnp.int32),
            pltpu.VMEM((RING,), jnp.int32),
            pltpu.VMEM((RING,), jnp.int32),
            pltpu.VMEM((K, H), jnp.float32),
            pltpu.VMEM((K, H), jnp.float32),
            pltpu.VMEM((L,), jnp.int32),
            pltpu.VMEM((L,), jnp.int32),
            pltpu.SemaphoreType.DMA,
            pltpu.SemaphoreType.DMA,
            pltpu.SemaphoreType.DMA,
            pltpu.SemaphoreType.DMA,
        ],
    )
    return kern(g, src, dst)


def _tc2_body(fi_ref, a0_ref, a1_ref, whh_ref, brnn_ref, h_ref):
    m = jnp.maximum(a0_ref[...], a1_ref[...])
    m = jnp.where(m == NEG_INF, 0.0, m)
    h_ref[...] = jnp.tanh(
        fi_ref[...]
        + jnp.dot(m, whh_ref[...], preferred_element_type=jnp.float32)
        + brnn_ref[...])


def _tc2(fi, a0, a1, W_hh, b_rnn):
    B = 1000
    grid = (N // B,)
    row_spec = pl.BlockSpec((B, H), lambda i: (i, 0))
    w_spec = pl.BlockSpec((H, H), lambda i: (0, 0))
    v_spec = pl.BlockSpec((1, H), lambda i: (0, 0))
    return pl.pallas_call(
        _tc2_body,
        grid=grid,
        in_specs=[row_spec, row_spec, row_spec, w_spec, v_spec],
        out_specs=row_spec,
        out_shape=jax.ShapeDtypeStruct((N, H), jnp.float32),
    )(fi, a0, a1, W_hh, b_rnn.reshape(1, H))


def kernel(x, edge_index, W_f, b_f, W_g, W_ih, W_hh, b_rnn):
    g, fi = _tc1(x, W_f, b_f, W_ih, b_rnn, W_g)
    src = edge_index[0]
    dst = edge_index[1]
    agg2 = _sc_segmax(g, src, dst)
    a0 = agg2[0].reshape(N, H)
    a1 = agg2[1].reshape(N, H)
    return _tc2(fi, a0, a1, W_hh, b_rnn)


# speedup vs baseline: 1.1127x; 1.0016x over previous
import functools

import jax
import jax.numpy as jnp
from jax import lax
from jax.experimental import pallas as pl
from jax.experimental.pallas import tpu as pltpu
from jax.experimental.pallas import tpu_sc as plsc

N = 10000
E = 320000
D = 128
H = 128

NC = 2
NS = 16
L = 16

R = N // NS
EC = E // NC
C = 2000
NCHUNK = EC // C
K = 128
NB = 2
RING = 4096
RMASK = RING - 1

NEG_INF = float("-inf")


def _tc1_body(x_ref, wf_ref, bf_ref, wih_ref, brnn_ref, wg_ref,
              g_ref, fi_ref):
    feat = jnp.tanh(
        jnp.dot(x_ref[...], wf_ref[...], preferred_element_type=jnp.float32)
        + bf_ref[...])
    fi = jnp.dot(feat, wih_ref[...], preferred_element_type=jnp.float32)
    h0 = jnp.tanh(fi + brnn_ref[...])
    g_ref[...] = jnp.dot(h0, wg_ref[...],
                         preferred_element_type=jnp.float32)
    fi_ref[...] = fi


def _tc1(x, W_f, b_f, W_ih, b_rnn, W_g):
    B = 1000
    grid = (N // B,)
    row_spec = pl.BlockSpec((B, H), lambda i: (i, 0))
    w_spec = pl.BlockSpec((H, H), lambda i: (0, 0))
    v_spec = pl.BlockSpec((1, H), lambda i: (0, 0))
    return pl.pallas_call(
        _tc1_body,
        grid=grid,
        in_specs=[pl.BlockSpec((B, D), lambda i: (i, 0)), w_spec, v_spec,
                  w_spec, v_spec, w_spec],
        out_specs=[row_spec, row_spec],
        out_shape=[jax.ShapeDtypeStruct((N, H), jnp.float32),
                   jax.ShapeDtypeStruct((N, H), jnp.float32)],
    )(x, W_f, b_f.reshape(1, H), W_ih, b_rnn.reshape(1, H), W_g)


def _sc_body(g_hbm, src_hbm, dst_hbm, out_hbm,
             acc, srcb0, dstb0, srcb1, dstb1, cidx, coff,
             rows0, rows1, stage_s, stage_o,
             se0, se1, sg0, sg1):
    cid = lax.axis_index("c")
    sid = lax.axis_index("s")
    lo = sid * R
    lane = lax.broadcasted_iota(jnp.int32, (L,), 0)
    ebase = cid * EC

    neg = jnp.full((L,), NEG_INF, dtype=jnp.float32)

    def init_body(i, _):
        for u in range(8):
            acc[pl.ds((i * 8 + u) * L, L)] = neg
        return 0

    lax.fori_loop(0, (R + 1) * H // (L * 8), init_body, 0)
    for i in range(((R + 1) * H // (L * 8)) * 8, (R + 1) * H // L):
        acc[pl.ds(i * L, L)] = neg

    chunk_bufs = ((srcb0, dstb0, se0), (srcb1, dstb1, se1))

    def chunk_descs(ch, par):
        sb, db, se = chunk_bufs[par]
        b = pl.multiple_of(ebase + ch * C, 8)
        c1 = pltpu.make_async_copy(src_hbm.at[pl.ds(b, C)], sb, se)
        c2 = pltpu.make_async_copy(dst_hbm.at[pl.ds(b, C)], db, se)
        return c1, c2

    def fire_chunk(ch, par):
        c1, c2 = chunk_descs(ch, par)
        c1.start()
        c2.start()

    def wait_chunk(ch, par):
        c1, c2 = chunk_descs(ch, par)
        c1.wait()
        c2.wait()

    gather_bufs = ((rows0, sg0), (rows1, sg1))

    def gather_desc(dlog, p):
        rws, sg = gather_bufs[p]
        dphys = pl.multiple_of(dlog & RMASK, K)
        return pltpu.make_async_copy(
            g_hbm.at[cidx.at[pl.ds(dphys, K)]], rws, sg)

    def fire_gather(dlog):
        pb = (dlog // K) & (NB - 1)
        for p in range(NB):
            @pl.when(pb == p)
            def _(p=p):
                gather_desc(dlog, p).start()

    SG = 4

    def _wait_acc(dlog, p):
        gather_desc(dlog, p).wait()
        rws = gather_bufs[p][0]
        dphys = pl.multiple_of(dlog & RMASK, K)

        def grp_body(jj, _):
            offv = coff[pl.ds(dphys + jj * L, L)]
            offs = [pl.multiple_of(offv[t], 8) for t in range(L)]
            dup = jnp.bool_(False)
            for t0 in range(0, L, SG):
                for a in range(SG):
                    for b in range(a + 1, SG):
                        dup = dup | (offs[t0 + a] == offs[t0 + b])

            @pl.when(jnp.logical_not(dup))
            def _fast():
                for t0 in range(0, L, SG):
                    vals = []
                    for u in range(SG):
                        r = jj * L + t0 + u
                        for c in range(H // L):
                            sl = pl.ds(offs[t0 + u] + c * L, L)
                            vals.append(jnp.maximum(
                                acc[sl], rws[r, pl.ds(c * L, L)]))
                    for u in range(SG):
                        for c in range(H // L):
                            sl = pl.ds(offs[t0 + u] + c * L, L)
                            acc[sl] = vals[u * (H // L) + c]

            @pl.when(dup)
            def _slow():
                for t in range(L):
                    r = jj * L + t
                    for c in range(H // L):
                        sl = pl.ds(offs[t] + c * L, L)
                        acc[sl] = jnp.maximum(
                            acc[sl], rws[r, pl.ds(c * L, L)])
            return 0

        lax.fori_loop(0, K // L, grp_body, 0)

    def acc_block(dlog):
        pb = (dlog // K) & (NB - 1)
        for p in range(NB):
            @pl.when(pb == p)
            def _(p=p):
                _wait_acc(dlog, p)

    def drains(cnt, dq, gq):
        def cond(carry):
            dq, gq = carry
            return cnt - dq >= K

        def body(carry):
            dq, gq = carry

            @pl.when(gq == dq)
            def _():
                fire_gather(dq)

            gq = jnp.maximum(gq, dq + K)
            for _ in range(NB - 1):
                ahead = (cnt - gq >= K) & (gq - dq < NB * K)

                @pl.when(ahead)
                def _():
                    fire_gather(gq)

                gq = jnp.where(ahead, gq + K, gq)
            acc_block(dq)
            return dq + K, gq

        return lax.while_loop(cond, body, (dq, gq))

    def filter_from(par, cnt_vec):
        sb, db, _ = chunk_bufs[par]

        FU = 5

        def filt_body(i, cnt_vec):
            for u in range(FU):
                dv = db[pl.ds((i * FU + u) * L, L)]
                sv = sb[pl.ds((i * FU + u) * L, L)]
                m = (dv >= lo) & (dv < lo + R)
                mi = jnp.where(m, jnp.int32(1), jnp.int32(0))
                pref = plsc.cumsum(mi)
                pos = (cnt_vec + pref - 1) & RMASK
                plsc.store_scatter(cidx, [pos], sv, mask=m)
                plsc.store_scatter(coff, [pos], (dv - lo) * H, mask=m)
                cnt_vec = cnt_vec + plsc.all_reduce_population_count(m)
            return cnt_vec

        return lax.fori_loop(0, C // L // FU, filt_body, cnt_vec)

    fire_chunk(jnp.int32(0), 0)

    def pair_body(i, carry):
        cnt_vec, dq, gq = carry
        ch0 = i * 2
        wait_chunk(ch0, 0)
        fire_chunk(ch0 + 1, 1)
        cnt_vec = filter_from(0, cnt_vec)
        dq, gq = drains(cnt_vec[0], dq, gq)
        wait_chunk(ch0 + 1, 1)

        @pl.when(ch0 + 2 < NCHUNK)
        def _():
            fire_chunk(ch0 + 2, 0)

        cnt_vec = filter_from(1, cnt_vec)
        dq, gq = drains(cnt_vec[0], dq, gq)
        return cnt_vec, dq, gq

    cnt_vec, dq, gq = lax.fori_loop(
        0, NCHUNK // 2, pair_body,
        (jnp.zeros((L,), jnp.int32), jnp.int32(0), jnp.int32(0)))
    cnt = cnt_vec[0]

    pad_idx = lane + sid * L
    pad_off = jnp.full((L,), R * H, dtype=jnp.int32)
    for t in range(K // L):
        pos = (cnt + t * L + lane) & RMASK
        plsc.store_scatter(cidx, [pos], pad_idx)
        plsc.store_scatter(coff, [pos], pad_off)
    drains(dq + K, dq, gq)

    pltpu.sync_copy(acc.at[pl.ds(0, R * H)],
                    out_hbm.at[cid, pl.ds(pl.multiple_of(lo * H, 8), R * H)])


def _sc_segmax(g, src, dst):
    mesh = plsc.VectorSubcoreMesh(core_axis_name="c", subcore_axis_name="s")
    kern = pl.kernel(
        _sc_body,
        out_type=jax.ShapeDtypeStruct((NC, N * H), jnp.float32),
        mesh=mesh,
        compiler_params=pltpu.CompilerParams(needs_layout_passes=False),
        scratch_types=[
            pltpu.VMEM(((R + 1) * H,), jnp.float32),
            pltpu.VMEM((C,), jnp.int32),
            pltpu.VMEM((C,), jnp.int32),
            pltpu.VMEM((C,), jnp.int32),
            pltpu.VMEM((C,), jnp.int32),
            pltpu.VMEM((RING,), jnp.int32),
            pltpu.VMEM((RING,), jnp.int32),
            pltpu.VMEM((K, H), jnp.float32),
            pltpu.VMEM((K, H), jnp.float32),
            pltpu.VMEM((L,), jnp.int32),
            pltpu.VMEM((L,), jnp.int32),
            pltpu.SemaphoreType.DMA,
            pltpu.SemaphoreType.DMA,
            pltpu.SemaphoreType.DMA,
            pltpu.SemaphoreType.DMA,
        ],
    )
    return kern(g, src, dst)


def _tc2_body(fi_ref, a0_ref, a1_ref, whh_ref, brnn_ref, h_ref):
    m = jnp.maximum(a0_ref[...], a1_ref[...])
    m = jnp.where(m == NEG_INF, 0.0, m)
    h_ref[...] = jnp.tanh(
        fi_ref[...]
        + jnp.dot(m, whh_ref[...], preferred_element_type=jnp.float32)
        + brnn_ref[...])


def _tc2(fi, a0, a1, W_hh, b_rnn):
    B = 1000
    grid = (N // B,)
    row_spec = pl.BlockSpec((B, H), lambda i: (i, 0))
    w_spec = pl.BlockSpec((H, H), lambda i: (0, 0))
    v_spec = pl.BlockSpec((1, H), lambda i: (0, 0))
    return pl.pallas_call(
        _tc2_body,
        grid=grid,
        in_specs=[row_spec, row_spec, row_spec, w_spec, v_spec],
        out_specs=row_spec,
        out_shape=jax.ShapeDtypeStruct((N, H), jnp.float32),
    )(fi, a0, a1, W_hh, b_rnn.reshape(1, H))


def kernel(x, edge_index, W_f, b_f, W_g, W_ih, W_hh, b_rnn):
    g, fi = _tc1(x, W_f, b_f, W_ih, b_rnn, W_g)
    src = edge_index[0]
    dst = edge_index[1]
    agg2 = _sc_segmax(g, src, dst)
    a0 = agg2[0].reshape(N, H)
    a1 = agg2[1].reshape(N, H)
    return _tc2(fi, a0, a1, W_hh, b_rnn)
